# Initial kernel scaffold; baseline (speedup 1.0000x reference)
import functools

import jax
import jax.numpy as jnp
from jax import lax
from jax.experimental import pallas as pl
from jax.experimental.pallas import tpu as pltpu
from jax.experimental.pallas import tpu_sc as plsc

NC = 2
NS = 16
NW = NC * NS
L = 16

N = 10000
E = 320000
D = 128
R = 8

CNTP = 81920
CSLICE = CNTP // NS
KC = 80
MB = 2000
ROWS_PER_TILE = N // NS
ZROWS = 25


def _sc_graph_kernel(src, dst, etype, zflat):
  mesh = plsc.VectorSubcoreMesh(core_axis_name="c", subcore_axis_name="s")

  @functools.partial(
      pl.kernel,
      out_type=jax.ShapeDtypeStruct((NC * N, D), jnp.float32),
      mesh=mesh,
      scratch_types=[
          pltpu.VMEM_SHARED((CNTP,), jnp.float32),
          pltpu.VMEM_SHARED((N, D), jnp.float32),
          pltpu.VMEM((MB,), jnp.int32),
          pltpu.VMEM((MB,), jnp.int32),
          pltpu.VMEM((MB,), jnp.int32),
          pltpu.VMEM((MB // KC, KC), jnp.int32),
          pltpu.VMEM((MB // KC, KC), jnp.int32),
          pltpu.VMEM((MB // KC, KC), jnp.int32),
          pltpu.VMEM((KC,), jnp.float32),
          pltpu.VMEM((KC,), jnp.float32),
          pltpu.VMEM((KC, D), jnp.float32),
          pltpu.VMEM((CNTP,), jnp.float32),
          pltpu.VMEM((CSLICE,), jnp.float32),
          pltpu.VMEM((ZROWS, D), jnp.float32),
          pltpu.SemaphoreType.DMA,
      ],
  )
  def k(src_hbm, dst_hbm, typ_hbm, z_hbm, parts_hbm,
        cnt_sp, acc_sp, srcb, dstb, typb, gb, hb, db,
        onesb, scaleb, rowsb, invb, sliceb, zerob, sem):
    cid = lax.axis_index("c")
    sid = lax.axis_index("s")
    zeros = jnp.zeros((L,), jnp.float32)

    @pl.loop(0, ZROWS)
    def _(i):
      for q in range(D // L):
        zerob[i, pl.ds(q * L, L)] = zeros

    @pl.loop(0, CSLICE // L)
    def _(i):
      sliceb[pl.ds(i * L, L)] = zeros

    @pl.loop(0, KC // L)
    def _(i):
      onesb[pl.ds(i * L, L)] = jnp.ones((L,), jnp.float32)

    pltpu.sync_copy(sliceb, cnt_sp.at[pl.ds(sid * CSLICE, CSLICE)])

    @pl.loop(0, ROWS_PER_TILE // ZROWS)
    def _(i):
      pltpu.sync_copy(
          zerob, acc_sp.at[pl.ds(sid * ROWS_PER_TILE + i * ZROWS, ZROWS)])

    plsc.subcore_barrier()

    base1 = sid * (E // NS)

    @pl.loop(0, E // NS // MB)
    def _(m):
      off = base1 + m * MB
      pltpu.sync_copy(dst_hbm.at[pl.ds(off, MB)], dstb)
      pltpu.sync_copy(typ_hbm.at[pl.ds(off, MB)], typb)

      @pl.loop(0, MB // KC)
      def _(c):
        for j in range(KC // L):
          sl = pl.ds(c * KC + j * L, L)
          hb[c, pl.ds(j * L, L)] = typb[sl] * N + dstb[sl]

      @pl.loop(0, MB // KC)
      def _(c):
        pltpu.sync_copy(onesb, cnt_sp.at[hb.at[c]], add=True)

    plsc.subcore_barrier()

    cbase = sid * CSLICE
    pltpu.sync_copy(cnt_sp.at[pl.ds(cbase, CSLICE)], sliceb)

    @pl.loop(0, CSLICE // L)
    def _(i):
      sl = pl.ds(i * L, L)
      sliceb[sl] = 1.0 / jnp.maximum(sliceb[sl], 1.0)

    pltpu.sync_copy(sliceb, cnt_sp.at[pl.ds(cbase, CSLICE)])
    plsc.subcore_barrier()

    pltpu.sync_copy(cnt_sp, invb)

    base2 = (cid * NS + sid) * (E // NW)

    @pl.loop(0, E // NW // MB)
    def _(m):
      off = base2 + m * MB
      pltpu.sync_copy(src_hbm.at[pl.ds(off, MB)], srcb)
      pltpu.sync_copy(dst_hbm.at[pl.ds(off, MB)], dstb)
      pltpu.sync_copy(typ_hbm.at[pl.ds(off, MB)], typb)

      @pl.loop(0, MB // KC)
      def _(c):
        for j in range(KC // L):
          sl = pl.ds(c * KC + j * L, L)
          t = typb[sl]
          hb[c, pl.ds(j * L, L)] = t * N + dstb[sl]
          gb[c, pl.ds(j * L, L)] = (t + 1) * N + srcb[sl]
          db[c, pl.ds(j * L, L)] = dstb[sl]

      @pl.loop(0, MB // KC)
      def _(c):
        for j in range(KC // L):
          scaleb[pl.ds(j * L, L)] = plsc.load_gather(
              invb, [hb[c, pl.ds(j * L, L)]])
        pltpu.async_copy(z_hbm.at[gb.at[c]], rowsb, sem).wait()

        @pl.loop(0, KC)
        def _(j):
          s = scaleb[j]
          for q in range(D // L):
            sl = pl.ds(q * L, L)
            rowsb[j, sl] = rowsb[j, sl] * s

        pltpu.sync_copy(rowsb, acc_sp.at[db.at[c]], add=True)

    plsc.subcore_barrier()

    rbase = sid * ROWS_PER_TILE
    pltpu.sync_copy(acc_sp.at[pl.ds(rbase, ROWS_PER_TILE)],
                    parts_hbm.at[pl.ds(cid * N + rbase, ROWS_PER_TILE)])

  return k(src, dst, etype, zflat)


BN = 2000


def _mm_body(x_ref, w_ref, b_ref, z_ref):
  i = pl.program_id(0)
  acc = jnp.dot(x_ref[...], w_ref[0], preferred_element_type=jnp.float32)
  sel = jnp.where(i == 0, 1.0, 0.0).astype(jnp.float32)
  z_ref[0] = acc + sel * b_ref[...]


def _fin_body(z0_ref, p_ref, a_ref, o_ref):
  o = z0_ref[...] + p_ref[0] + p_ref[1]
  o_ref[...] = jnp.where(o > 0, o, a_ref[...] * o)


def kernel(x, edge_index, edge_type, weight, root, bias, prelu_a):
  src = edge_index[0]
  dst = edge_index[1]
  wcat = jnp.concatenate([root[None], weight], axis=0)

  zfull = pl.pallas_call(
      _mm_body,
      grid=(R + 1, N // BN),
      in_specs=[
          pl.BlockSpec((BN, D), lambda i, nb: (nb, 0)),
          pl.BlockSpec((1, D, D), lambda i, nb: (i, 0, 0)),
          pl.BlockSpec((1, D), lambda i, nb: (0, 0)),
      ],
      out_specs=pl.BlockSpec((1, BN, D), lambda i, nb: (i, nb, 0)),
      out_shape=jax.ShapeDtypeStruct((R + 1, N, D), jnp.float32),
  )(x, wcat, bias[None])

  zflat = zfull.reshape((R + 1) * N, D)
  parts = _sc_graph_kernel(src, dst, edge_type, zflat).reshape(NC, N, D)

  return pl.pallas_call(
      _fin_body,
      grid=(N // BN,),
      in_specs=[
          pl.BlockSpec((BN, D), lambda nb: (nb, 0)),
          pl.BlockSpec((NC, BN, D), lambda nb: (0, nb, 0)),
          pl.BlockSpec((1, D), lambda nb: (0, 0)),
      ],
      out_specs=pl.BlockSpec((BN, D), lambda nb: (nb, 0)),
      out_shape=jax.ShapeDtypeStruct((N, D), jnp.float32),
  )(zfull[0], parts, prelu_a[None])


# speedup vs baseline: 21.6518x; 21.6518x over previous
import functools

import jax
import jax.numpy as jnp
from jax import lax
from jax.experimental import pallas as pl
from jax.experimental.pallas import tpu as pltpu
from jax.experimental.pallas import tpu_sc as plsc

NC = 2
NS = 16
NW = NC * NS
L = 16

N = 10000
E = 320000
D = 128
R = 8

CNTP = 81920
CSLICE = CNTP // NS
KC = 80
MB = 2000
NP = 10240
ROWS_PER_TILE = NP // NS
ZROWS = 40


def _sc_graph_kernel(src, dst, etype, zflat):
  mesh = plsc.VectorSubcoreMesh(core_axis_name="c", subcore_axis_name="s")

  @functools.partial(
      pl.kernel,
      out_type=jax.ShapeDtypeStruct((NC * NP, D), jnp.float32),
      mesh=mesh,
      scratch_types=[
          pltpu.VMEM_SHARED((CNTP,), jnp.float32),
          pltpu.VMEM_SHARED((NP, D), jnp.float32),
          pltpu.VMEM((MB,), jnp.int32),
          pltpu.VMEM((MB,), jnp.int32),
          pltpu.VMEM((MB,), jnp.int32),
          pltpu.VMEM((MB // KC, KC), jnp.int32),
          pltpu.VMEM((MB // KC, KC), jnp.int32),
          pltpu.VMEM((MB // KC, KC), jnp.int32),
          pltpu.VMEM((KC,), jnp.float32),
          pltpu.VMEM((KC,), jnp.float32),
          pltpu.VMEM((KC, D), jnp.float32),
          pltpu.VMEM((CSLICE,), jnp.float32),
          pltpu.VMEM((ZROWS, D), jnp.float32),
          pltpu.SemaphoreType.DMA,
      ],
  )
  def k(src_hbm, dst_hbm, typ_hbm, z_hbm, parts_hbm,
        cnt_sp, acc_sp, srcb, dstb, typb, gb, hb, db,
        onesb, scaleb, rowsb, sliceb, zerob, sem):
    cid = lax.axis_index("c")
    sid = lax.axis_index("s")
    zeros = jnp.zeros((L,), jnp.float32)

    @pl.loop(0, ZROWS)
    def _(i):
      for q in range(D // L):
        zerob[i, pl.ds(q * L, L)] = zeros

    @pl.loop(0, CSLICE // L)
    def _(i):
      sliceb[pl.ds(i * L, L)] = zeros

    @pl.loop(0, KC // L)
    def _(i):
      onesb[pl.ds(i * L, L)] = jnp.ones((L,), jnp.float32)

    pltpu.sync_copy(sliceb, cnt_sp.at[pl.ds(sid * CSLICE, CSLICE)])

    @pl.loop(0, ROWS_PER_TILE // ZROWS)
    def _(i):
      pltpu.sync_copy(
          zerob, acc_sp.at[pl.ds(sid * ROWS_PER_TILE + i * ZROWS, ZROWS)])

    plsc.subcore_barrier()

    base1 = sid * (E // NS)

    @pl.loop(0, E // NS // MB)
    def _(m):
      off = base1 + m * MB
      pltpu.sync_copy(dst_hbm.at[pl.ds(off, MB)], dstb)
      pltpu.sync_copy(typ_hbm.at[pl.ds(off, MB)], typb)

      @pl.loop(0, MB // KC)
      def _(c):
        for j in range(KC // L):
          sl = pl.ds(c * KC + j * L, L)
          hb[c, pl.ds(j * L, L)] = typb[sl] * N + dstb[sl]

      @pl.loop(0, MB // KC)
      def _(c):
        pltpu.sync_copy(onesb, cnt_sp.at[hb.at[c]], add=True)

    plsc.subcore_barrier()

    cbase = sid * CSLICE
    pltpu.sync_copy(cnt_sp.at[pl.ds(cbase, CSLICE)], sliceb)

    @pl.loop(0, CSLICE // L)
    def _(i):
      sl = pl.ds(i * L, L)
      sliceb[sl] = 1.0 / jnp.maximum(sliceb[sl], 1.0)

    pltpu.sync_copy(sliceb, cnt_sp.at[pl.ds(cbase, CSLICE)])
    plsc.subcore_barrier()

    base2 = (cid * NS + sid) * (E // NW)

    @pl.loop(0, E // NW // MB)
    def _(m):
      off = base2 + m * MB
      pltpu.sync_copy(src_hbm.at[pl.ds(off, MB)], srcb)
      pltpu.sync_copy(dst_hbm.at[pl.ds(off, MB)], dstb)
      pltpu.sync_copy(typ_hbm.at[pl.ds(off, MB)], typb)

      @pl.loop(0, MB // KC)
      def _(c):
        for j in range(KC // L):
          sl = pl.ds(c * KC + j * L, L)
          t = typb[sl]
          hb[c, pl.ds(j * L, L)] = t * N + dstb[sl]
          gb[c, pl.ds(j * L, L)] = (t + 1) * N + srcb[sl]
          db[c, pl.ds(j * L, L)] = dstb[sl]

      @pl.loop(0, MB // KC)
      def _(c):
        pltpu.async_copy(cnt_sp.at[hb.at[c]], scaleb, sem).wait()
        pltpu.async_copy(z_hbm.at[gb.at[c]], rowsb, sem).wait()

        @pl.loop(0, KC // L)
        def _(g):
          sv = scaleb[pl.ds(g * L, L)]
          for j in range(L):
            s = sv[j]
            row = g * L + j
            for q in range(D // L):
              sl = pl.ds(q * L, L)
              rowsb[row, sl] = rowsb[row, sl] * s

        pltpu.sync_copy(rowsb, acc_sp.at[db.at[c]], add=True)

    plsc.subcore_barrier()

    rbase = sid * ROWS_PER_TILE
    pltpu.sync_copy(acc_sp.at[pl.ds(rbase, ROWS_PER_TILE)],
                    parts_hbm.at[pl.ds(cid * NP + rbase, ROWS_PER_TILE)])

  return k(src, dst, etype, zflat)


BN = 2000


def _mm_body(x_ref, w_ref, b_ref, z_ref):
  i = pl.program_id(0)
  acc = jnp.dot(x_ref[...], w_ref[0], preferred_element_type=jnp.float32)
  sel = jnp.where(i == 0, 1.0, 0.0).astype(jnp.float32)
  z_ref[0] = acc + sel * b_ref[...]


def _fin_body(z0_ref, p_ref, a_ref, o_ref):
  o = z0_ref[...] + p_ref[0] + p_ref[1]
  o_ref[...] = jnp.where(o > 0, o, a_ref[...] * o)


def kernel(x, edge_index, edge_type, weight, root, bias, prelu_a):
  src = edge_index[0]
  dst = edge_index[1]
  wcat = jnp.concatenate([root[None], weight], axis=0)

  zfull = pl.pallas_call(
      _mm_body,
      grid=(R + 1, N // BN),
      in_specs=[
          pl.BlockSpec((BN, D), lambda i, nb: (nb, 0)),
          pl.BlockSpec((1, D, D), lambda i, nb: (i, 0, 0)),
          pl.BlockSpec((1, D), lambda i, nb: (0, 0)),
      ],
      out_specs=pl.BlockSpec((1, BN, D), lambda i, nb: (i, nb, 0)),
      out_shape=jax.ShapeDtypeStruct((R + 1, N, D), jnp.float32),
  )(x, wcat, bias[None])

  zflat = zfull.reshape((R + 1) * N, D)
  parts = _sc_graph_kernel(src, dst, edge_type, zflat)
  parts = parts.reshape(NC, NP, D)[:, :N]

  return pl.pallas_call(
      _fin_body,
      grid=(N // BN,),
      in_specs=[
          pl.BlockSpec((BN, D), lambda nb: (nb, 0)),
          pl.BlockSpec((NC, BN, D), lambda nb: (0, nb, 0)),
          pl.BlockSpec((1, D), lambda nb: (0, 0)),
      ],
      out_specs=pl.BlockSpec((BN, D), lambda nb: (nb, 0)),
      out_shape=jax.ShapeDtypeStruct((N, D), jnp.float32),
  )(zfull[0], parts, prelu_a[None])


# speedup vs baseline: 29.4951x; 1.3622x over previous
import functools

import jax
import jax.numpy as jnp
from jax import lax
from jax.experimental import pallas as pl
from jax.experimental.pallas import tpu as pltpu
from jax.experimental.pallas import tpu_sc as plsc

NC = 2
NS = 16
NW = NC * NS
L = 16

N = 10000
E = 320000
D = 128
R = 8

CNTP = 81920
CSLICE = CNTP // NS
KC = 80
NBUF = 2
EPT = E // NW
ECT = E // NS
MB = 2000
MC = MB // KC
CHALF = CSLICE // 2
NP = 10240
ROWS_PER_TILE = NP // NS
ZROWS = 16


def _sc_graph_kernel(src, dst, etype, zflat):
  mesh = plsc.VectorSubcoreMesh(core_axis_name="c", subcore_axis_name="s")

  @functools.partial(
      pl.kernel,
      out_type=jax.ShapeDtypeStruct((NC * NP, D), jnp.float32),
      mesh=mesh,
      scratch_types=[
          pltpu.VMEM_SHARED((CNTP,), jnp.float32),
          pltpu.VMEM_SHARED((NP, D), jnp.float32),
          pltpu.VMEM((MB,), jnp.int32),
          pltpu.VMEM((MB,), jnp.int32),
          pltpu.VMEM((MB,), jnp.int32),
          pltpu.VMEM((MC, KC), jnp.int32),
          pltpu.VMEM((KC,), jnp.float32),
          [pltpu.VMEM((KC,), jnp.float32)] * NBUF,
          [pltpu.VMEM((KC, D), jnp.float32)] * NBUF,
          pltpu.VMEM((CHALF,), jnp.float32),
          pltpu.VMEM((ZROWS, D), jnp.float32),
          [pltpu.SemaphoreType.DMA] * NBUF,
          [pltpu.SemaphoreType.DMA] * NBUF,
          pltpu.SemaphoreType.DMA,
          pltpu.SemaphoreType.DMA,
      ],
  )
  def k(src_hbm, dst_hbm, typ_hbm, z_hbm, parts_hbm,
        cnt_sp, acc_sp, srcb, dstb, typb, db,
        onesb, scalebs, rowsbs, sliceb, zerob,
        gsems, ssems, wsem, esem):
    cid = lax.axis_index("c")
    sid = lax.axis_index("s")
    zeros = jnp.zeros((L,), jnp.float32)

    @pl.loop(0, ZROWS)
    def _(i):
      for q in range(D // L):
        zerob[i, pl.ds(q * L, L)] = zeros

    @pl.loop(0, CHALF // L)
    def _(i):
      sliceb[pl.ds(i * L, L)] = zeros

    @pl.loop(0, KC // L)
    def _(i):
      onesb[pl.ds(i * L, L)] = jnp.ones((L,), jnp.float32)

    for p in range(2):
      pltpu.sync_copy(sliceb, cnt_sp.at[pl.ds(sid * CSLICE + p * CHALF, CHALF)])

    @pl.loop(0, ROWS_PER_TILE // ZROWS)
    def _(i):
      pltpu.sync_copy(
          zerob, acc_sp.at[pl.ds(sid * ROWS_PER_TILE + i * ZROWS, ZROWS)])

    plsc.subcore_barrier()

    base1 = sid * ECT

    @pl.loop(0, ECT // MB)
    def _(m):
      off = base1 + m * MB
      f1 = pltpu.async_copy(dst_hbm.at[pl.ds(off, MB)], dstb, esem)
      f2 = pltpu.async_copy(typ_hbm.at[pl.ds(off, MB)], typb, esem)
      f1.wait()
      f2.wait()

      @pl.loop(0, MC)
      def _(c):
        for j in range(KC // L):
          sl = pl.ds(c * KC + j * L, L)
          db[c, pl.ds(j * L, L)] = typb[sl] * N + dstb[sl]

      @pl.loop(0, MC // 5)
      def _(w):
        descs = [
            pltpu.async_copy(onesb, cnt_sp.at[db.at[w * 5 + b]], wsem)
            for b in range(5)
        ]
        for d_ in descs:
          d_.wait()

    plsc.subcore_barrier()

    cbase = sid * CSLICE
    for p in range(2):
      pltpu.sync_copy(cnt_sp.at[pl.ds(cbase + p * CHALF, CHALF)], sliceb)

      @pl.loop(0, CHALF // L)
      def _(i):
        sl = pl.ds(i * L, L)
        sliceb[sl] = 1.0 / jnp.maximum(sliceb[sl], 1.0)

      pltpu.sync_copy(sliceb, cnt_sp.at[pl.ds(cbase + p * CHALF, CHALF)])
    plsc.subcore_barrier()

    base2 = (cid * NS + sid) * EPT

    def chunk_in(c, b):
      ebase = c * KC
      sd = pltpu.async_copy(
          cnt_sp.at[typb.at[pl.ds(ebase, KC)]], scalebs[b], ssems[b])
      rd = pltpu.async_copy(
          z_hbm.at[srcb.at[pl.ds(ebase, KC)]], rowsbs[b], gsems[b])
      return sd, rd

    def chunk_out(c, b, sd, rd):
      sd.wait()
      rd.wait()
      rowsb = rowsbs[b]
      scaleb = scalebs[b]

      @pl.loop(0, KC // L)
      def _(gg):
        sv = scaleb[pl.ds(gg * L, L)]
        for j in range(L):
          s = sv[j]
          row = gg * L + j
          for q in range(D // L):
            sl = pl.ds(q * L, L)
            rowsb[row, sl] = rowsb[row, sl] * s

      return pltpu.async_copy(rowsb, acc_sp.at[db.at[c]], wsem)

    @pl.loop(0, EPT // MB)
    def _(m):
      off = base2 + m * MB
      f1 = pltpu.async_copy(src_hbm.at[pl.ds(off, MB)], srcb, esem)
      f2 = pltpu.async_copy(dst_hbm.at[pl.ds(off, MB)], dstb, esem)
      f3 = pltpu.async_copy(typ_hbm.at[pl.ds(off, MB)], typb, esem)
      f1.wait()
      f2.wait()
      f3.wait()

      @pl.loop(0, MC)
      def _(c):
        for j in range(KC // L):
          sl = pl.ds(c * KC + j * L, L)
          t = typb[sl]
          d_ = dstb[sl]
          srcb[sl] = (t + 1) * N + srcb[sl]
          typb[sl] = t * N + d_
          db[c, pl.ds(j * L, L)] = d_

      @pl.loop(0, (MC - 1) // NBUF)
      def _(w):
        c0 = w * NBUF
        ins = [chunk_in(c0 + b, b) for b in range(NBUF)]
        outs = [chunk_out(c0 + b, b, *ins[b]) for b in range(NBUF)]
        for d_ in outs:
          d_.wait()

      sd, rd = chunk_in(MC - 1, 0)
      chunk_out(MC - 1, 0, sd, rd).wait()

    plsc.subcore_barrier()

    rbase = sid * ROWS_PER_TILE
    pltpu.sync_copy(acc_sp.at[pl.ds(rbase, ROWS_PER_TILE)],
                    parts_hbm.at[pl.ds(cid * NP + rbase, ROWS_PER_TILE)])

  return k(src, dst, etype, zflat)


BN = 2000


def _mm_body(x_ref, w_ref, b_ref, z_ref):
  i = pl.program_id(0)
  acc = jnp.dot(x_ref[...], w_ref[0], preferred_element_type=jnp.float32)
  sel = jnp.where(i == 0, 1.0, 0.0).astype(jnp.float32)
  z_ref[0] = acc + sel * b_ref[...]


def _fin_body(z0_ref, p_ref, a_ref, o_ref):
  o = z0_ref[...] + p_ref[0] + p_ref[1]
  o_ref[...] = jnp.where(o > 0, o, a_ref[...] * o)


def kernel(x, edge_index, edge_type, weight, root, bias, prelu_a):
  src = edge_index[0]
  dst = edge_index[1]
  wcat = jnp.concatenate([root[None], weight], axis=0)

  zfull = pl.pallas_call(
      _mm_body,
      grid=(R + 1, N // BN),
      in_specs=[
          pl.BlockSpec((BN, D), lambda i, nb: (nb, 0)),
          pl.BlockSpec((1, D, D), lambda i, nb: (i, 0, 0)),
          pl.BlockSpec((1, D), lambda i, nb: (0, 0)),
      ],
      out_specs=pl.BlockSpec((1, BN, D), lambda i, nb: (i, nb, 0)),
      out_shape=jax.ShapeDtypeStruct((R + 1, N, D), jnp.float32),
  )(x, wcat, bias[None])

  zflat = zfull.reshape((R + 1) * N, D)
  parts = _sc_graph_kernel(src, dst, edge_type, zflat)
  parts = parts.reshape(NC, NP, D)[:, :N]

  return pl.pallas_call(
      _fin_body,
      grid=(N // BN,),
      in_specs=[
          pl.BlockSpec((BN, D), lambda nb: (nb, 0)),
          pl.BlockSpec((NC, BN, D), lambda nb: (0, nb, 0)),
          pl.BlockSpec((1, D), lambda nb: (0, 0)),
      ],
      out_specs=pl.BlockSpec((BN, D), lambda nb: (nb, 0)),
      out_shape=jax.ShapeDtypeStruct((N, D), jnp.float32),
  )(zfull[0], parts, prelu_a[None])
